# Initial kernel scaffold; baseline (speedup 1.0000x reference)
import jax
import jax.numpy as jnp
from jax.experimental import pallas as pl


def kernel(species, charges, edge_src, edge_dst, distances, switch):
    raise NotImplementedError("write your pallas kernel here")


# speedup vs baseline: 147.5915x; 147.5915x over previous
import functools

import jax
import jax.numpy as jnp
from jax import lax
from jax.experimental import pallas as pl
from jax.experimental.pallas import tpu as pltpu
from jax.experimental.pallas import tpu_sc as plsc

BOHR = 0.52917721067121
NC, NS = 2, 16
NW = NC * NS
ROW = 128
CR = 78


@functools.lru_cache(maxsize=None)
def _make_sc_kernel(N, E):
    assert E % ROW == 0
    R = E // ROW
    base_rows = R // NW
    X = R % NW
    Np = ((N + 1023) // 1024) * 1024
    SL = Np // NS
    n_full, rem = divmod(base_rows, CR)
    CE = CR * ROW

    mesh = plsc.VectorSubcoreMesh(
        core_axis_name="c", subcore_axis_name="s",
        num_cores=NC, num_subcores=NS)

    @functools.partial(
        pl.kernel,
        out_type=jax.ShapeDtypeStruct((NC, Np), jnp.float32),
        mesh=mesh,
        compiler_params=pltpu.CompilerParams(use_tc_tiling_on_sc=False,
                                              needs_layout_passes=False),
        scratch_types=[
            pltpu.VMEM((N,), jnp.float32),
            pltpu.VMEM((CR, ROW), jnp.int32),
            pltpu.VMEM((CE,), jnp.int32),
            pltpu.VMEM((CE,), jnp.float32),
            pltpu.VMEM((CE,), jnp.float32),
            pltpu.VMEM((CE,), jnp.float32),
            pltpu.VMEM_SHARED((Np,), jnp.float32),
            pltpu.SemaphoreType.DMA,
        ],
    )
    def sc_kernel(q_hbm, src_hbm, dst_hbm, dist_hbm, sw_hbm, out_hbm,
                  q_v, src_v, dst_v, dist_v, sw_v, val_v, accum, sem):
        cid = lax.axis_index("c")
        sid = lax.axis_index("s")
        wid = sid * NC + cid

        pltpu.sync_copy(q_hbm, q_v)

        def _zero(i, c):
            val_v[pl.ds(i * 16, 16)] = jnp.zeros((16,), jnp.float32)
            return c
        lax.fori_loop(0, SL // 16, _zero, 0)
        pltpu.sync_copy(val_v.at[pl.ds(0, SL)], accum.at[pl.ds(sid * SL, SL)])
        plsc.subcore_barrier()

        def do_chunk(r0, nrows):
            ne = nrows * ROW
            pltpu.sync_copy(src_hbm.at[pl.ds(r0, nrows)], src_v.at[pl.ds(0, nrows)])
            pltpu.sync_copy(dst_hbm.at[pl.ds(r0 * ROW, ne)], dst_v.at[pl.ds(0, ne)])
            pltpu.sync_copy(dist_hbm.at[pl.ds(r0 * ROW, ne)], dist_v.at[pl.ds(0, ne)])
            pltpu.sync_copy(sw_hbm.at[pl.ds(r0 * ROW, ne)], sw_v.at[pl.ds(0, ne)])

            def _compute(j, c):
                sl = pl.ds(j * 16, 16)
                qd = plsc.load_gather(q_v, [dst_v[sl]])
                val_v[sl] = sw_v[sl] * (BOHR / dist_v[sl]) * qd
                return c
            lax.fori_loop(0, ne // 16, _compute, 0)

            descs = [
                pltpu.async_copy(val_v.at[pl.ds(r * ROW, ROW)],
                                 accum.at[src_v.at[r]], sem, add=True)
                for r in range(nrows)
            ]
            for d in descs:
                d.wait()

        row0 = wid * base_rows
        for c in range(n_full):
            do_chunk(row0 + c * CR, CR)
        if rem:
            do_chunk(row0 + n_full * CR, rem)
        if X:
            @pl.when(wid < X)
            def _extra():
                do_chunk(NW * base_rows + wid, 1)

        plsc.subcore_barrier()
        pltpu.sync_copy(accum.at[pl.ds(sid * SL, SL)], val_v.at[pl.ds(0, SL)])
        pltpu.sync_copy(val_v.at[pl.ds(0, SL)], out_hbm.at[cid, pl.ds(sid * SL, SL)])

    return sc_kernel, Np


def _combine_body(q_ref, p_ref, o_ref):
    o_ref[...] = 0.5 * q_ref[...] * (p_ref[0] + p_ref[1])


def kernel(species, charges, edge_src, edge_dst, distances, switch):
    del species
    N = charges.shape[0]
    E = edge_src.shape[0]
    Ep = -(-E // ROW) * ROW
    if Ep != E:
        pad = Ep - E
        edge_src = jnp.pad(edge_src, (0, pad))
        edge_dst = jnp.pad(edge_dst, (0, pad))
        distances = jnp.pad(distances, (0, pad), constant_values=1.0)
        switch = jnp.pad(switch, (0, pad))

    sc_kernel, Np = _make_sc_kernel(N, Ep)
    src2d = edge_src.astype(jnp.int32).reshape(-1, ROW)
    partial = sc_kernel(
        charges,
        src2d,
        edge_dst.astype(jnp.int32),
        distances,
        switch,
    )

    qp = jnp.pad(charges, (0, Np - N)).reshape(Np // 128, 128)
    out = pl.pallas_call(
        _combine_body,
        out_shape=jax.ShapeDtypeStruct((Np // 128, 128), jnp.float32),
    )(qp, partial.reshape(NC, Np // 128, 128))
    return out.reshape(-1)[:N]
